# baseline (speedup 1.0000x reference)
import functools

import jax
import jax.numpy as jnp
from jax import lax
from jax.experimental import pallas as pl
from jax.experimental.pallas import tpu as pltpu
from jax.experimental.pallas import tpu_sc as plsc


@functools.cache
def _make_sc_gather(V, D, B):
  info = plsc.get_sparse_core_info()
  NC, NS = info.num_cores, info.num_subcores
  NW = NC * NS
  assert B % (8 * NW) == 0 and D % info.num_lanes == 0
  b_per_w = B // NW
  mesh = plsc.VectorSubcoreMesh(core_axis_name="c", subcore_axis_name="s")

  @functools.partial(
      pl.kernel,
      mesh=mesh,
      out_type=jax.ShapeDtypeStruct((B, D), jnp.float32),
      scratch_types=[
          pltpu.VMEM((b_per_w,), jnp.int32),
          pltpu.VMEM((b_per_w, D), jnp.float32),
          pltpu.SemaphoreType.DMA,
      ],
  )
  def sc_gather(table_hbm, idx_hbm, out_hbm, idx_v, rows_v, sem):
    wid = lax.axis_index("s") * NC + lax.axis_index("c")
    base = wid * b_per_w
    pltpu.sync_copy(idx_hbm.at[pl.ds(base, b_per_w)], idx_v)
    pltpu.async_copy(table_hbm.at[idx_v], rows_v, sem).wait()
    pltpu.sync_copy(rows_v, out_hbm.at[pl.ds(base, b_per_w)])

  return sc_gather


_TV = 2048


def _mm_body(x_ref, w_ref, o_ref):
  o_ref[...] = lax.dot_general(
      w_ref[...], x_ref[...], (((1,), (1,)), ((), ())),
      preferred_element_type=jnp.float32)


def _matmul_t(x_bf, w):
  T, D = x_bf.shape
  V = w.shape[0]
  return pl.pallas_call(
      _mm_body,
      grid=(pl.cdiv(V, _TV),),
      in_specs=[
          pl.BlockSpec((T, D), lambda i: (0, 0)),
          pl.BlockSpec((_TV, D), lambda i: (i, 0)),
      ],
      out_specs=pl.BlockSpec((_TV, T), lambda i: (i, 0)),
      out_shape=jax.ShapeDtypeStruct((V, T), jnp.float32),
      compiler_params=pltpu.CompilerParams(
          vmem_limit_bytes=100 * 1024 * 1024),
  )(x_bf, w)


def kernel(tokens, w):
  B, T = tokens.shape
  V, D = w.shape
  idx = tokens.reshape(B * T)
  x = _make_sc_gather(V, D, B * T)(w, idx)
  x_bf = x.astype(jnp.bfloat16)
  logits_t = _matmul_t(x_bf, w)
  return logits_t.T.reshape(B, T, V)


# speedup vs baseline: 1.0020x; 1.0020x over previous
import functools

import jax
import jax.numpy as jnp
from jax import lax
from jax.experimental import pallas as pl
from jax.experimental.pallas import tpu as pltpu
from jax.experimental.pallas import tpu_sc as plsc


@functools.cache
def _make_sc_gather(V, D, B):
  info = plsc.get_sparse_core_info()
  NC, NS = info.num_cores, info.num_subcores
  NW = NC * NS
  assert B % (8 * NW) == 0 and D % info.num_lanes == 0
  b_per_w = B // NW
  mesh = plsc.VectorSubcoreMesh(core_axis_name="c", subcore_axis_name="s")

  @functools.partial(
      pl.kernel,
      mesh=mesh,
      out_type=jax.ShapeDtypeStruct((B, D), jnp.float32),
      scratch_types=[
          pltpu.VMEM((b_per_w,), jnp.int32),
          pltpu.VMEM((b_per_w, D), jnp.float32),
          pltpu.SemaphoreType.DMA,
      ],
  )
  def sc_gather(table_hbm, idx_hbm, out_hbm, idx_v, rows_v, sem):
    wid = lax.axis_index("s") * NC + lax.axis_index("c")
    base = wid * b_per_w
    pltpu.sync_copy(idx_hbm.at[pl.ds(base, b_per_w)], idx_v)
    pltpu.async_copy(table_hbm.at[idx_v], rows_v, sem).wait()
    pltpu.sync_copy(rows_v, out_hbm.at[pl.ds(base, b_per_w)])

  return sc_gather


_TV = 2000


def _mm_body(x_ref, w_ref, o_ref):
  o_ref[...] = lax.dot_general(
      w_ref[...], x_ref[...], (((1,), (1,)), ((), ())),
      preferred_element_type=jnp.float32)


def _matmul_t(x_bf, w):
  T, D = x_bf.shape
  V = w.shape[0]
  return pl.pallas_call(
      _mm_body,
      grid=(V // _TV,),
      in_specs=[
          pl.BlockSpec((T, D), lambda i: (0, 0)),
          pl.BlockSpec((_TV, D), lambda i: (i, 0)),
      ],
      out_specs=pl.BlockSpec((_TV, T), lambda i: (i, 0)),
      out_shape=jax.ShapeDtypeStruct((V, T), jnp.float32),
      compiler_params=pltpu.CompilerParams(
          vmem_limit_bytes=100 * 1024 * 1024),
  )(x_bf, w)


def kernel(tokens, w):
  B, T = tokens.shape
  V, D = w.shape
  idx = tokens.reshape(B * T)
  x = _make_sc_gather(V, D, B * T)(w, idx)
  x_bf = x.astype(jnp.bfloat16)
  logits_t = _matmul_t(x_bf, w)
  return logits_t.T.reshape(B, T, V)
